# Initial kernel scaffold; baseline (speedup 1.0000x reference)
import jax
import jax.numpy as jnp
from jax.experimental import pallas as pl


def kernel(t, x, edge_index):
    raise NotImplementedError("write your pallas kernel here")


# speedup vs baseline: 59.3200x; 59.3200x over previous
import functools

import jax
import jax.numpy as jnp
from jax import lax
from jax.experimental import pallas as pl
from jax.experimental.pallas import tpu as pltpu
from jax.experimental.pallas import tpu_sc as plsc

_N = 100000
_E = 1600000
_D = 6.0

_LANES = 128
_NPAD = 100352
_ROWS_X = _NPAD // _LANES

_NC = 2
_NS = 16
_NW = _NC * _NS

_KB = 24
_NB = 17
_RPW = _KB * _NB
_EROWS = _NW * _RPW
_EPAD = _EROWS * _LANES

_SLICE = _NPAD // _NS


def _sc_spmm(x_pad, src2d, dst2d, zeros):
    mesh = plsc.VectorSubcoreMesh(core_axis_name="c", subcore_axis_name="s")

    @functools.partial(
        pl.kernel,
        mesh=mesh,
        out_type=jax.ShapeDtypeStruct((_NC * _NPAD,), jnp.float32),
        scratch_types=[
            pltpu.VMEM((_KB, _LANES), jnp.int32),
            pltpu.VMEM((_KB, _LANES), jnp.int32),
            pltpu.VMEM((_KB, _LANES), jnp.float32),
            pltpu.VMEM_SHARED((_NPAD,), jnp.float32),
            pltpu.VMEM_SHARED((_NPAD,), jnp.float32),
            pltpu.SemaphoreType.DMA,
            pltpu.SemaphoreType.DMA,
        ],
    )
    def k(x_hbm, src_hbm, dst_hbm, zeros_hbm, out_hbm,
          srcb, dstb, gathb, xs, acc, gsem, ssem):
        cid = lax.axis_index("c")
        sid = lax.axis_index("s")

        pltpu.sync_copy(zeros_hbm.at[pl.ds(sid * _SLICE, _SLICE)],
                        acc.at[pl.ds(sid * _SLICE, _SLICE)])
        pltpu.sync_copy(x_hbm.at[pl.ds(sid * _SLICE, _SLICE)],
                        xs.at[pl.ds(sid * _SLICE, _SLICE)])
        plsc.subcore_barrier()

        w = cid * _NS + sid
        row0 = w * _RPW

        def block(b, _):
            rb = row0 + b * _KB
            pltpu.sync_copy(src_hbm.at[pl.ds(rb, _KB)], srcb)
            pltpu.sync_copy(dst_hbm.at[pl.ds(rb, _KB)], dstb)
            gds = [pltpu.async_copy(xs.at[srcb.at[j]], gathb.at[j], gsem)
                   for j in range(_KB)]
            for d in gds:
                d.wait()
            sds = [pltpu.async_copy(gathb.at[j], acc.at[dstb.at[j]], ssem,
                                    add=True)
                   for j in range(_KB)]
            for d in sds:
                d.wait()
            return ()

        lax.fori_loop(0, _NB, block, (), unroll=False)
        plsc.subcore_barrier()

        pltpu.sync_copy(acc.at[pl.ds(sid * _SLICE, _SLICE)],
                        out_hbm.at[pl.ds(cid * _NPAD + sid * _SLICE, _SLICE)])

    return k(x_pad, src2d, dst2d, zeros)


def _tc_combine(x2d, partials):
    def body(x_ref, p_ref, o_ref):
        xx = x_ref[...]
        ax = p_ref[0] + p_ref[1]
        o_ref[...] = (-_D) * xx + (1.0 - xx) * ax

    return pl.pallas_call(
        body,
        out_shape=jax.ShapeDtypeStruct((_ROWS_X, _LANES), jnp.float32),
    )(x2d, partials)


def kernel(t, x, edge_index):
    del t
    x_flat = x[:, 0]
    x_pad = jnp.pad(x_flat, (0, _NPAD - _N))
    src = jnp.pad(edge_index[0], (0, _EPAD - _E))
    dst = jnp.pad(edge_index[1], (0, _EPAD - _E), constant_values=_N)
    src2d = src.reshape(_EROWS, _LANES)
    dst2d = dst.reshape(_EROWS, _LANES)
    zeros = jnp.zeros((_NPAD,), jnp.float32)

    partials = _sc_spmm(x_pad, src2d, dst2d, zeros)
    out2d = _tc_combine(x_pad.reshape(_ROWS_X, _LANES),
                        partials.reshape(_NC, _ROWS_X, _LANES))
    return out2d.reshape(-1)[:_N].reshape(_N, 1)
